# baseline (speedup 1.0000x reference)
import jax
import jax.numpy as jnp
from jax.experimental import pallas as pl
from jax.experimental.pallas import tpu as pltpu

_GRID = 8


def _route(lt):
    def row(k):
        return lt[k:k + 1, :]

    part = row(1) > row(0)
    bv = row(2)
    bi = jnp.zeros_like(bv, dtype=jnp.int32)
    t = row(3) > bv
    bi = jnp.where(t, 1, bi)
    bv = jnp.where(t, row(3), bv)
    t = row(4) > bv
    bi = jnp.where(t, 2, bi)
    rv = row(5)
    ri = jnp.zeros_like(bv, dtype=jnp.int32)
    for k in range(1, 8):
        t = row(5 + k) > rv
        ri = jnp.where(t, k, ri)
        rv = jnp.where(t, row(5 + k), rv)
    a0 = row(13) >= row(14)

    branch = jnp.where(bi == 0, 3, jnp.where(bi == 1, ri + 1, jnp.where(a0, 4, 6)))
    return jnp.where(part, branch, 0).astype(jnp.int32)


def _body(xt_ref, w1_ref, w2_ref, w3r_ref, w3a_ref,
          b1_ref, b2_ref, b3r_ref, b3a_ref, o_ref):
    wcat = jnp.concatenate(
        [w1_ref[...], w2_ref[...], w3r_ref[...], w3a_ref[...]], axis=1)
    lt = jax.lax.dot_general(
        wcat, xt_ref[...], (((0,), (0,)), ((), ())),
        preferred_element_type=jnp.float32)
    bias = jnp.concatenate(
        [jnp.full((1, 1), b_ref[k], jnp.float32)
         for b_ref, n in ((b1_ref, 2), (b2_ref, 3), (b3r_ref, 8), (b3a_ref, 2))
         for k in range(n)], axis=0)
    lt = lt + bias
    final = _route(lt)
    o_ref[...] = final.reshape(o_ref.shape)


def kernel(x, W1, b1, W2, b2, W3r, b3r, W3a, b3a):
    batch = x.shape[0]
    d = x.size // batch
    xt = jnp.transpose(x, (1, 2, 3, 0)).reshape(d, batch)

    bs = batch // _GRID
    out = pl.pallas_call(
        _body,
        grid=(_GRID,),
        in_specs=[
            pl.BlockSpec((d, bs), lambda i: (0, i)),
            pl.BlockSpec((d, 2), lambda i: (0, 0)),
            pl.BlockSpec((d, 3), lambda i: (0, 0)),
            pl.BlockSpec((d, 8), lambda i: (0, 0)),
            pl.BlockSpec((d, 2), lambda i: (0, 0)),
            pl.BlockSpec(memory_space=pltpu.SMEM),
            pl.BlockSpec(memory_space=pltpu.SMEM),
            pl.BlockSpec(memory_space=pltpu.SMEM),
            pl.BlockSpec(memory_space=pltpu.SMEM),
        ],
        out_specs=pl.BlockSpec((bs,), lambda i: (i,)),
        out_shape=jax.ShapeDtypeStruct((batch,), jnp.int32),
    )(xt, W1, W2, W3r, W3a, b1, b2, b3r, b3a)
    return out


# speedup vs baseline: 1.2186x; 1.2186x over previous
import jax
import jax.numpy as jnp
from jax.experimental import pallas as pl
from jax.experimental.pallas import tpu as pltpu

_GRID = 8
_NS = 2


def _route(lt):
    def row(k):
        return lt[k:k + 1, :]

    part = row(1) > row(0)
    bv = row(2)
    bi = jnp.zeros_like(bv, dtype=jnp.int32)
    t = row(3) > bv
    bi = jnp.where(t, 1, bi)
    bv = jnp.where(t, row(3), bv)
    t = row(4) > bv
    bi = jnp.where(t, 2, bi)
    rv = row(5)
    ri = jnp.zeros_like(bv, dtype=jnp.int32)
    for k in range(1, 8):
        t = row(5 + k) > rv
        ri = jnp.where(t, k, ri)
        rv = jnp.where(t, row(5 + k), rv)
    a0 = row(13) >= row(14)

    branch = jnp.where(bi == 0, 3, jnp.where(bi == 1, ri + 1, jnp.where(a0, 4, 6)))
    return jnp.where(part, branch, 0).astype(jnp.int32)


def _body(x0_ref, x1_ref, wt_ref, bc_ref, o_ref):
    i = pl.program_id(0)
    bias = jnp.concatenate(
        [jnp.full((1, 1), bc_ref[k], jnp.float32) for k in range(15)]
        + [jnp.zeros((113, 1), jnp.float32)], axis=0)
    for s, x_ref in enumerate((x0_ref, x1_ref)):
        n = x_ref.shape[1]
        lt = jnp.dot(wt_ref[...], x_ref[...],
                     preferred_element_type=jnp.float32)
        lt = lt + bias
        final = _route(lt)
        base = (i * _NS + s) * n
        o_ref[pl.ds(base, n)] = final.reshape(n)


def kernel(x, W1, b1, W2, b2, W3r, b3r, W3a, b3a):
    batch = x.shape[0]
    d = x.size // batch
    xt = jnp.transpose(x, (1, 2, 3, 0)).reshape(d, batch)
    W = jnp.concatenate([W1, W2, W3r, W3a], axis=1)
    Wt = jnp.pad(W, ((0, 0), (0, 128 - W.shape[1]))).T
    bc = jnp.concatenate([b1, b2, b3r, b3a], axis=0)

    bs = batch // (_GRID * _NS)
    out = pl.pallas_call(
        _body,
        grid=(_GRID,),
        in_specs=[
            pl.BlockSpec((d, bs), lambda i: (0, _NS * i)),
            pl.BlockSpec((d, bs), lambda i: (0, _NS * i + 1)),
            pl.BlockSpec((128, d), lambda i: (0, 0)),
            pl.BlockSpec(memory_space=pltpu.SMEM),
        ],
        out_specs=pl.BlockSpec((batch,), lambda i: (0,)),
        out_shape=jax.ShapeDtypeStruct((batch,), jnp.int32),
    )(xt, xt, Wt, bc)
    return out
